# Initial kernel scaffold; baseline (speedup 1.0000x reference)
import jax
import jax.numpy as jnp
from jax.experimental import pallas as pl


def kernel(x, W1_rel, b1, W1_root, W2_rel, b2, W2_root, Wmu_rel, bmu, Wmu_root, Wls_rel, bls, Wls_root, edge_index):
    raise NotImplementedError("write your pallas kernel here")


# speedup vs baseline: 10.4484x; 10.4484x over previous
import functools

import jax
import jax.numpy as jnp
from jax import lax
from jax.experimental import pallas as pl
from jax.experimental.pallas import tpu as pltpu
from jax.experimental.pallas import tpu_sc as plsc

N_NODES = 10000
N_EDGES = 320000

NC, NS = 2, 16
NW = NC * NS
CHUNK = 128
EDGES_PER_TILE = 10112
NCHUNKS = EDGES_PER_TILE // CHUNK
E_PAD = NW * EDGES_PER_TILE
N_ACC = 10112
ZROWS = N_ACC // NS
WB_ROWS = 624


def _make_sc_agg(D):
  mesh = plsc.VectorSubcoreMesh(core_axis_name="c", subcore_axis_name="s")

  @functools.partial(
      pl.kernel,
      out_type=jax.ShapeDtypeStruct((NC, N_NODES, D), jnp.float32),
      mesh=mesh,
      scratch_types=[
          pltpu.VMEM((NCHUNKS, CHUNK), jnp.int32),
          pltpu.VMEM((NCHUNKS, CHUNK), jnp.int32),
          pltpu.VMEM((CHUNK, D), jnp.float32),
          pltpu.VMEM_SHARED((N_ACC, D), jnp.float32),
          pltpu.SemaphoreType.DMA,
      ],
      compiler_params=pltpu.CompilerParams(use_tc_tiling_on_sc=False),
  )
  def agg(table, srcp, dstp, zrows, out, src_v, dst_v, rows_v, acc, sem):
    c = lax.axis_index("c")
    s = lax.axis_index("s")
    wid = c * NS + s

    pltpu.sync_copy(zrows, acc.at[pl.ds(s * ZROWS, ZROWS)])
    pltpu.sync_copy(srcp.at[wid], src_v)
    pltpu.sync_copy(dstp.at[wid], dst_v)
    plsc.subcore_barrier()

    def body(i, carry):
      pltpu.async_copy(table.at[src_v.at[i]], rows_v, sem).wait()
      pltpu.sync_copy(rows_v, acc.at[dst_v.at[i]], add=True)
      return carry

    lax.fori_loop(0, NCHUNKS, body, 0)
    plsc.subcore_barrier()

    pltpu.sync_copy(acc.at[pl.ds(s * WB_ROWS, WB_ROWS)],
                    out.at[c, pl.ds(s * WB_ROWS, WB_ROWS)])

    @pl.when(s == NS - 1)
    def _tail():
      base = NS * WB_ROWS
      pltpu.sync_copy(acc.at[pl.ds(base, N_NODES - base)],
                      out.at[c, pl.ds(base, N_NODES - base)])

  return agg


_sc_agg_144 = _make_sc_agg(144)
_sc_agg_128 = _make_sc_agg(128)
_sc_agg_16 = _make_sc_agg(16)


_TC_BLK = 2000
_GRID = N_NODES // _TC_BLK


def _tc1_body(s1_ref, x_ref, w1r_ref, b1_ref, w1t_ref, w2r_ref, w2t_ref,
              b2_ref, p2_ref, r2_ref, ic_ref):
  s = s1_ref[0] + s1_ref[1]
  ic = 1.0 / jnp.maximum(s[:, 128:136], 1.0)
  agg = s[:, :128] * ic[:, :1]
  h1 = jnp.maximum(
      jnp.dot(agg, w1r_ref[...], preferred_element_type=jnp.float32)
      + b1_ref[...]
      + jnp.dot(x_ref[...], w1t_ref[...], preferred_element_type=jnp.float32),
      0.0)
  p2_ref[...] = jnp.dot(h1, w2r_ref[...], preferred_element_type=jnp.float32)
  r2_ref[...] = (
      jnp.dot(h1, w2t_ref[...], preferred_element_type=jnp.float32)
      + b2_ref[...])
  ic_ref[...] = ic


def _tc2_body(s2_ref, r2_ref, ic_ref, w3r_ref, w3t_ref, b3_ref,
              p3_ref, r3_ref):
  h2 = jnp.maximum(
      (s2_ref[0] + s2_ref[1]) * ic_ref[:, :1] + r2_ref[...], 0.0)
  p3_ref[...] = jnp.dot(h2, w3r_ref[...], preferred_element_type=jnp.float32)
  r3_ref[...] = (
      jnp.dot(h2, w3t_ref[...], preferred_element_type=jnp.float32)
      + b3_ref[...])


def _tc3_body(s3_ref, r3_ref, ic_ref, out_ref):
  out_ref[...] = (s3_ref[0] + s3_ref[1]) * ic_ref[:, :1] + r3_ref[...]


def _row_blk(shape_tail):
  return pl.BlockSpec((_TC_BLK,) + shape_tail, lambda i: (i,) + (0,) * len(shape_tail))


def _part_blk(d):
  return pl.BlockSpec((NC, _TC_BLK, d), lambda i: (0, i, 0))


def _full_blk(shape):
  return pl.BlockSpec(shape, lambda i: (0,) * len(shape))


def kernel(x, W1_rel, b1, W1_root, W2_rel, b2, W2_root, Wmu_rel, bmu,
           Wmu_root, Wls_rel, bls, Wls_root, edge_index):
  src = edge_index[0].astype(jnp.int32)
  dst = edge_index[1].astype(jnp.int32)
  pad = E_PAD - N_EDGES
  ar = jnp.arange(pad, dtype=jnp.int32)
  srcp = jnp.concatenate([src, ar % N_NODES]).reshape(NW, NCHUNKS, CHUNK)
  dstp = jnp.concatenate(
      [dst, N_NODES + (ar % (N_ACC - N_NODES))]).reshape(NW, NCHUNKS, CHUNK)

  x_aug = jnp.concatenate(
      [x, jnp.ones((N_NODES, 16), jnp.float32)], axis=1)
  z144 = jnp.zeros((ZROWS, 144), jnp.float32)
  z128 = jnp.zeros((ZROWS, 128), jnp.float32)
  z16 = jnp.zeros((ZROWS, 16), jnp.float32)

  w3_rel = jnp.zeros((128, 16), jnp.float32)
  w3_rel = w3_rel.at[:, 0:2].set(Wmu_rel).at[:, 2:4].set(Wls_rel)
  w3_root = jnp.zeros((128, 16), jnp.float32)
  w3_root = w3_root.at[:, 0:2].set(Wmu_root).at[:, 2:4].set(Wls_root)
  b3 = jnp.zeros((1, 16), jnp.float32)
  b3 = b3.at[0, 0:2].set(bmu).at[0, 2:4].set(bls)

  s1 = _sc_agg_144(x_aug, srcp, dstp, z144)

  p2, r2, ic = pl.pallas_call(
      _tc1_body,
      grid=(_GRID,),
      in_specs=[
          _part_blk(144),
          _row_blk((128,)),
          _full_blk((128, 256)),
          _full_blk((1, 256)),
          _full_blk((128, 256)),
          _full_blk((256, 128)),
          _full_blk((256, 128)),
          _full_blk((1, 128)),
      ],
      out_specs=[_row_blk((128,)), _row_blk((128,)), _row_blk((8,))],
      out_shape=[
          jax.ShapeDtypeStruct((N_NODES, 128), jnp.float32),
          jax.ShapeDtypeStruct((N_NODES, 128), jnp.float32),
          jax.ShapeDtypeStruct((N_NODES, 8), jnp.float32),
      ],
  )(s1, x, W1_rel, b1.reshape(1, 256), W1_root, W2_rel, W2_root,
    b2.reshape(1, 128))

  s2 = _sc_agg_128(p2, srcp, dstp, z128)

  p3, r3 = pl.pallas_call(
      _tc2_body,
      grid=(_GRID,),
      in_specs=[
          _part_blk(128),
          _row_blk((128,)),
          _row_blk((8,)),
          _full_blk((128, 16)),
          _full_blk((128, 16)),
          _full_blk((1, 16)),
      ],
      out_specs=[_row_blk((16,)), _row_blk((16,))],
      out_shape=[
          jax.ShapeDtypeStruct((N_NODES, 16), jnp.float32),
          jax.ShapeDtypeStruct((N_NODES, 16), jnp.float32),
      ],
  )(s2, r2, ic, w3_rel, w3_root, b3)

  s3 = _sc_agg_16(p3, srcp, dstp, z16)

  out = pl.pallas_call(
      _tc3_body,
      grid=(_GRID,),
      in_specs=[_part_blk(16), _row_blk((16,)), _row_blk((8,))],
      out_specs=_row_blk((16,)),
      out_shape=jax.ShapeDtypeStruct((N_NODES, 16), jnp.float32),
  )(s3, r3, ic)

  return out[:, 0:2], out[:, 2:4]


# speedup vs baseline: 16.1586x; 1.5465x over previous
import functools

import jax
import jax.numpy as jnp
from jax import lax
from jax.experimental import pallas as pl
from jax.experimental.pallas import tpu as pltpu
from jax.experimental.pallas import tpu_sc as plsc

N_NODES = 10000
N_EDGES = 320000

NC, NS = 2, 16
NW = NC * NS
CHUNK = 128
EDGES_PER_TILE = 10240
NCHUNKS = EDGES_PER_TILE // CHUNK
E_PAD = NW * EDGES_PER_TILE
N_ACC = 10112
ZROWS = N_ACC // NS
WB_ROWS = 624


def _zero_acc(zrows, acc, s):
  pltpu.sync_copy(zrows, acc.at[pl.ds(s * ZROWS, ZROWS)])


def _write_back(acc, out, c, s):
  pltpu.sync_copy(acc.at[pl.ds(s * WB_ROWS, WB_ROWS)],
                  out.at[c, pl.ds(s * WB_ROWS, WB_ROWS)])

  @pl.when(s == NS - 1)
  def _tail():
    base = NS * WB_ROWS
    pltpu.sync_copy(acc.at[pl.ds(base, N_NODES - base)],
                    out.at[c, pl.ds(base, N_NODES - base)])


def _make_sc_cnt():
  mesh = plsc.VectorSubcoreMesh(core_axis_name="c", subcore_axis_name="s")

  @functools.partial(
      pl.kernel,
      out_type=jax.ShapeDtypeStruct((NC, N_NODES, 16), jnp.float32),
      mesh=mesh,
      scratch_types=[
          pltpu.VMEM((NCHUNKS, CHUNK), jnp.int32),
          pltpu.VMEM((CHUNK, 16), jnp.float32),
          pltpu.VMEM_SHARED((N_ACC, 16), jnp.float32),
          pltpu.SemaphoreType.DMA,
      ],
      compiler_params=pltpu.CompilerParams(use_tc_tiling_on_sc=False),
  )
  def cnt(ones_blk, dstp, zrows, out, dst_v, ones_v, acc, sem):
    c = lax.axis_index("c")
    s = lax.axis_index("s")
    wid = c * NS + s

    _zero_acc(zrows, acc, s)
    pltpu.sync_copy(dstp.at[wid], dst_v)
    pltpu.sync_copy(ones_blk, ones_v)
    plsc.subcore_barrier()

    K = 16

    def outer(io, carry):
      base = io * K
      for k in range(K):
        pltpu.async_copy(ones_v, acc.at[dst_v.at[base + k]], sem, add=True)
      for k in range(K):
        pltpu.make_async_copy(ones_v, acc.at[dst_v.at[base + k]], sem).wait()
      return carry

    lax.fori_loop(0, NCHUNKS // K, outer, 0)
    plsc.subcore_barrier()
    _write_back(acc, out, c, s)

  return cnt


def _make_sc_agg(D, nbuf, half_staged):
  mesh = plsc.VectorSubcoreMesh(core_axis_name="c", subcore_axis_name="s")
  stage = NCHUNKS // 2 if half_staged else NCHUNKS

  @functools.partial(
      pl.kernel,
      out_type=jax.ShapeDtypeStruct((NC, N_NODES, D), jnp.float32),
      mesh=mesh,
      scratch_types=[
          pltpu.VMEM((stage, CHUNK), jnp.int32),
          pltpu.VMEM((stage, CHUNK), jnp.int32),
          [pltpu.VMEM((CHUNK, D), jnp.float32) for _ in range(nbuf)],
          pltpu.VMEM_SHARED((N_ACC, D), jnp.float32),
          [pltpu.SemaphoreType.DMA] * nbuf,
      ],
      compiler_params=pltpu.CompilerParams(use_tc_tiling_on_sc=False),
  )
  def agg(table, srcp, dstp, zrows, out, src_v, dst_v, rows_v, acc, sems):
    c = lax.axis_index("c")
    s = lax.axis_index("s")
    wid = c * NS + s

    _zero_acc(zrows, acc, s)

    def load_idx(half):
      pltpu.sync_copy(srcp.at[wid, pl.ds(half * stage, stage)], src_v)
      pltpu.sync_copy(dstp.at[wid, pl.ds(half * stage, stage)], dst_v)

    load_idx(0)
    plsc.subcore_barrier()

    def run_chunks(lo, hi):
      for b in range(nbuf):
        pltpu.async_copy(table.at[src_v.at[lo + b]], rows_v[b], sems[b])

      def outer(io, carry):
        for b in range(nbuf):
          i = lo + io * nbuf + b
          pltpu.make_async_copy(table.at[src_v.at[i]], rows_v[b],
                                sems[b]).wait()
          pltpu.sync_copy(rows_v[b], acc.at[dst_v.at[i]], add=True)

          @pl.when(i + nbuf < hi)
          def _refill():
            pltpu.async_copy(table.at[src_v.at[i + nbuf]], rows_v[b],
                             sems[b])
        return carry

      lax.fori_loop(0, (hi - lo) // nbuf, outer, 0)

    run_chunks(0, stage)
    if half_staged:
      load_idx(1)
      run_chunks(0, stage)

    plsc.subcore_barrier()
    _write_back(acc, out, c, s)

  return agg


_sc_cnt = _make_sc_cnt()
_sc_agg_128 = _make_sc_agg(128, nbuf=2, half_staged=True)
_sc_agg_16 = _make_sc_agg(16, nbuf=4, half_staged=False)


_TC_BLK = 2000
_GRID = N_NODES // _TC_BLK


def _tc1_body(s1_ref, cnt_ref, x_ref, w1r_ref, b1_ref, w1t_ref, w2r_ref,
              w2t_ref, b2_ref, p2_ref, r2_ref, ic_ref):
  ic = 1.0 / jnp.maximum(cnt_ref[0, :, :8] + cnt_ref[1, :, :8], 1.0)
  agg = (s1_ref[0] + s1_ref[1]) * ic[:, :1]
  h1 = jnp.maximum(
      jnp.dot(agg, w1r_ref[...], preferred_element_type=jnp.float32)
      + b1_ref[...]
      + jnp.dot(x_ref[...], w1t_ref[...], preferred_element_type=jnp.float32),
      0.0)
  p2_ref[...] = jnp.dot(h1, w2r_ref[...], preferred_element_type=jnp.float32)
  r2_ref[...] = (
      jnp.dot(h1, w2t_ref[...], preferred_element_type=jnp.float32)
      + b2_ref[...])
  ic_ref[...] = ic


def _tc2_body(s2_ref, r2_ref, ic_ref, w3r_ref, w3t_ref, b3_ref,
              p3_ref, r3_ref):
  h2 = jnp.maximum(
      (s2_ref[0] + s2_ref[1]) * ic_ref[:, :1] + r2_ref[...], 0.0)
  p3_ref[...] = jnp.dot(h2, w3r_ref[...], preferred_element_type=jnp.float32)
  r3_ref[...] = (
      jnp.dot(h2, w3t_ref[...], preferred_element_type=jnp.float32)
      + b3_ref[...])


def _tc3_body(s3_ref, r3_ref, ic_ref, out_ref):
  out_ref[...] = (s3_ref[0] + s3_ref[1]) * ic_ref[:, :1] + r3_ref[...]


def _row_blk(shape_tail):
  return pl.BlockSpec((_TC_BLK,) + shape_tail,
                      lambda i: (i,) + (0,) * len(shape_tail))


def _part_blk(d):
  return pl.BlockSpec((NC, _TC_BLK, d), lambda i: (0, i, 0))


def _full_blk(shape):
  return pl.BlockSpec(shape, lambda i: (0,) * len(shape))


def kernel(x, W1_rel, b1, W1_root, W2_rel, b2, W2_root, Wmu_rel, bmu,
           Wmu_root, Wls_rel, bls, Wls_root, edge_index):
  src = edge_index[0].astype(jnp.int32)
  dst = edge_index[1].astype(jnp.int32)
  pad = E_PAD - N_EDGES
  ar = jnp.arange(pad, dtype=jnp.int32)
  srcp = jnp.concatenate([src, ar % N_NODES]).reshape(NW, NCHUNKS, CHUNK)
  dstp = jnp.concatenate(
      [dst, N_NODES + (ar % (N_ACC - N_NODES))]).reshape(NW, NCHUNKS, CHUNK)

  ones_blk = jnp.ones((CHUNK, 16), jnp.float32)
  z128 = jnp.zeros((ZROWS, 128), jnp.float32)
  z16 = jnp.zeros((ZROWS, 16), jnp.float32)

  w3_rel = jnp.zeros((128, 16), jnp.float32)
  w3_rel = w3_rel.at[:, 0:2].set(Wmu_rel).at[:, 2:4].set(Wls_rel)
  w3_root = jnp.zeros((128, 16), jnp.float32)
  w3_root = w3_root.at[:, 0:2].set(Wmu_root).at[:, 2:4].set(Wls_root)
  b3 = jnp.zeros((1, 16), jnp.float32)
  b3 = b3.at[0, 0:2].set(bmu).at[0, 2:4].set(bls)

  cnt = _sc_cnt(ones_blk, dstp, z16)
  s1 = _sc_agg_128(x, srcp, dstp, z128)

  p2, r2, ic = pl.pallas_call(
      _tc1_body,
      grid=(_GRID,),
      in_specs=[
          _part_blk(128),
          _part_blk(16),
          _row_blk((128,)),
          _full_blk((128, 256)),
          _full_blk((1, 256)),
          _full_blk((128, 256)),
          _full_blk((256, 128)),
          _full_blk((256, 128)),
          _full_blk((1, 128)),
      ],
      out_specs=[_row_blk((128,)), _row_blk((128,)), _row_blk((8,))],
      out_shape=[
          jax.ShapeDtypeStruct((N_NODES, 128), jnp.float32),
          jax.ShapeDtypeStruct((N_NODES, 128), jnp.float32),
          jax.ShapeDtypeStruct((N_NODES, 8), jnp.float32),
      ],
  )(s1, cnt, x, W1_rel, b1.reshape(1, 256), W1_root, W2_rel, W2_root,
    b2.reshape(1, 128))

  s2 = _sc_agg_128(p2, srcp, dstp, z128)

  p3, r3 = pl.pallas_call(
      _tc2_body,
      grid=(_GRID,),
      in_specs=[
          _part_blk(128),
          _row_blk((128,)),
          _row_blk((8,)),
          _full_blk((128, 16)),
          _full_blk((128, 16)),
          _full_blk((1, 16)),
      ],
      out_specs=[_row_blk((16,)), _row_blk((16,))],
      out_shape=[
          jax.ShapeDtypeStruct((N_NODES, 16), jnp.float32),
          jax.ShapeDtypeStruct((N_NODES, 16), jnp.float32),
      ],
  )(s2, r2, ic, w3_rel, w3_root, b3)

  s3 = _sc_agg_16(p3, srcp, dstp, z16)

  out = pl.pallas_call(
      _tc3_body,
      grid=(_GRID,),
      in_specs=[_part_blk(16), _row_blk((16,)), _row_blk((8,))],
      out_specs=_row_blk((16,)),
      out_shape=jax.ShapeDtypeStruct((N_NODES, 16), jnp.float32),
  )(s3, r3, ic)

  return out[:, 0:2], out[:, 2:4]


# speedup vs baseline: 16.6499x; 1.0304x over previous
import functools

import jax
import jax.numpy as jnp
from jax import lax
from jax.experimental import pallas as pl
from jax.experimental.pallas import tpu as pltpu
from jax.experimental.pallas import tpu_sc as plsc

N_NODES = 10000
N_EDGES = 320000

NC, NS = 2, 16
NW = NC * NS
CHUNK = 128
EDGES_PER_TILE = 10240
NCHUNKS = EDGES_PER_TILE // CHUNK
E_PAD = NW * EDGES_PER_TILE
N_ACC = 10112
ZROWS = N_ACC // NS
WB_ROWS = 624


def _write_back(acc, out, c, s):
  pltpu.sync_copy(acc.at[pl.ds(s * WB_ROWS, WB_ROWS)],
                  out.at[c, pl.ds(s * WB_ROWS, WB_ROWS)])

  @pl.when(s == NS - 1)
  def _tail():
    base = NS * WB_ROWS
    pltpu.sync_copy(acc.at[pl.ds(base, N_NODES - base)],
                    out.at[c, pl.ds(base, N_NODES - base)])


def _make_sc_agg(D, nbuf, n_stage, with_counts):
  mesh = plsc.VectorSubcoreMesh(core_axis_name="c", subcore_axis_name="s")
  stage = NCHUNKS // n_stage

  out_type = [jax.ShapeDtypeStruct((NC, N_NODES, D), jnp.float32)]
  scratch = [
      pltpu.VMEM((stage, CHUNK), jnp.int32),
      pltpu.VMEM((stage, CHUNK), jnp.int32),
      [pltpu.VMEM((CHUNK, D), jnp.float32) for _ in range(nbuf)],
      pltpu.VMEM_SHARED((N_ACC, D), jnp.float32),
      [pltpu.SemaphoreType.DMA] * nbuf,
  ]
  if with_counts:
    out_type.append(jax.ShapeDtypeStruct((NC, N_NODES, 16), jnp.float32))
    scratch += [
        pltpu.VMEM((CHUNK, 16), jnp.float32),
        pltpu.VMEM_SHARED((N_ACC, 16), jnp.float32),
        pltpu.SemaphoreType.DMA,
    ]

  @functools.partial(
      pl.kernel,
      out_type=out_type,
      mesh=mesh,
      scratch_types=scratch,
      compiler_params=pltpu.CompilerParams(use_tc_tiling_on_sc=False),
  )
  def agg(table, srcp, dstp, zrows, *rest):
    if with_counts:
      (zrows16, out, out_c, src_v, dst_v, rows_v, acc, sems,
       ones_v, acc_c, sem_c) = rest
    else:
      out, src_v, dst_v, rows_v, acc, sems = rest
    c = lax.axis_index("c")
    s = lax.axis_index("s")
    wid = c * NS + s

    pltpu.sync_copy(zrows, acc.at[pl.ds(s * ZROWS, ZROWS)])
    if with_counts:
      pltpu.sync_copy(zrows16, acc_c.at[pl.ds(s * ZROWS, ZROWS)])

      def fill_ones(r, carry):
        ones_v[r] = jnp.ones((16,), jnp.float32)
        return carry

      lax.fori_loop(0, CHUNK, fill_ones, 0)

    def load_idx(st):
      pltpu.sync_copy(srcp.at[wid, pl.ds(st * stage, stage)], src_v)
      pltpu.sync_copy(dstp.at[wid, pl.ds(st * stage, stage)], dst_v)

    load_idx(0)
    plsc.subcore_barrier()

    def run_chunks():
      for b in range(nbuf):
        pltpu.async_copy(table.at[src_v.at[b]], rows_v[b], sems[b])

      def outer(io, carry):
        for b in range(nbuf):
          i = io * nbuf + b
          pltpu.make_async_copy(table.at[src_v.at[i]], rows_v[b],
                                sems[b]).wait()
          pltpu.sync_copy(rows_v[b], acc.at[dst_v.at[i]], add=True)
          if with_counts:
            pltpu.async_copy(ones_v, acc_c.at[dst_v.at[i]], sem_c, add=True)

          @pl.when(i + nbuf < stage)
          def _refill():
            pltpu.async_copy(table.at[src_v.at[i + nbuf]], rows_v[b],
                             sems[b])
        return carry

      lax.fori_loop(0, stage // nbuf, outer, 0)
      if with_counts:
        def drain(io, carry):
          pltpu.make_async_copy(ones_v, acc_c.at[dst_v.at[0]], sem_c).wait()
          return carry
        lax.fori_loop(0, stage, drain, 0)

    run_chunks()
    for st in range(1, n_stage):
      load_idx(st)
      run_chunks()

    plsc.subcore_barrier()
    _write_back(acc, out, c, s)
    if with_counts:
      _write_back(acc_c, out_c, c, s)

  return agg


_sc_agg_x_cnt = _make_sc_agg(128, nbuf=2, n_stage=4, with_counts=True)
_sc_agg_128 = _make_sc_agg(128, nbuf=2, n_stage=2, with_counts=False)
_sc_agg_16 = _make_sc_agg(16, nbuf=8, n_stage=1, with_counts=False)


_TC_BLK = 2000
_GRID = N_NODES // _TC_BLK


def _tc1_body(s1_ref, cnt_ref, x_ref, w1r_ref, b1_ref, w1t_ref, w2r_ref,
              w2t_ref, b2_ref, p2_ref, r2_ref, ic_ref):
  ic = 1.0 / jnp.maximum(cnt_ref[0, :, :8] + cnt_ref[1, :, :8], 1.0)
  agg = (s1_ref[0] + s1_ref[1]) * ic[:, :1]
  h1 = jnp.maximum(
      jnp.dot(agg, w1r_ref[...], preferred_element_type=jnp.float32)
      + b1_ref[...]
      + jnp.dot(x_ref[...], w1t_ref[...], preferred_element_type=jnp.float32),
      0.0)
  p2_ref[...] = jnp.dot(h1, w2r_ref[...], preferred_element_type=jnp.float32)
  r2_ref[...] = (
      jnp.dot(h1, w2t_ref[...], preferred_element_type=jnp.float32)
      + b2_ref[...])
  ic_ref[...] = ic


def _tc2_body(s2_ref, r2_ref, ic_ref, wmr_ref, wlr_ref, wmt_ref, wlt_ref,
              bm_ref, bl_ref, p3_ref, r3_ref):
  h2 = jnp.maximum(
      (s2_ref[0] + s2_ref[1]) * ic_ref[:, :1] + r2_ref[...], 0.0)
  w3r = jnp.concatenate(
      [wmr_ref[...], wlr_ref[...],
       jnp.zeros((128, 12), jnp.float32)], axis=1)
  w3t = jnp.concatenate(
      [wmt_ref[...], wlt_ref[...],
       jnp.zeros((128, 12), jnp.float32)], axis=1)
  b3 = jnp.concatenate(
      [bm_ref[...], bl_ref[...], jnp.zeros((1, 12), jnp.float32)], axis=1)
  p3_ref[...] = jnp.dot(h2, w3r, preferred_element_type=jnp.float32)
  r3_ref[...] = jnp.dot(h2, w3t, preferred_element_type=jnp.float32) + b3


def _tc3_body(s3_ref, r3_ref, ic_ref, out_ref):
  out_ref[...] = (s3_ref[0] + s3_ref[1]) * ic_ref[:, :1] + r3_ref[...]


def _row_blk(shape_tail):
  return pl.BlockSpec((_TC_BLK,) + shape_tail,
                      lambda i: (i,) + (0,) * len(shape_tail))


def _part_blk(d):
  return pl.BlockSpec((NC, _TC_BLK, d), lambda i: (0, i, 0))


def _full_blk(shape):
  return pl.BlockSpec(shape, lambda i: (0,) * len(shape))


def kernel(x, W1_rel, b1, W1_root, W2_rel, b2, W2_root, Wmu_rel, bmu,
           Wmu_root, Wls_rel, bls, Wls_root, edge_index):
  src = edge_index[0].astype(jnp.int32)
  dst = edge_index[1].astype(jnp.int32)
  pad = E_PAD - N_EDGES
  ar = jnp.arange(pad, dtype=jnp.int32)
  srcp = jnp.concatenate([src, ar % N_NODES]).reshape(NW, NCHUNKS, CHUNK)
  dstp = jnp.concatenate(
      [dst, N_NODES + (ar % (N_ACC - N_NODES))]).reshape(NW, NCHUNKS, CHUNK)

  z128 = jnp.zeros((ZROWS, 128), jnp.float32)
  z16 = jnp.zeros((ZROWS, 16), jnp.float32)

  s1, cnt = _sc_agg_x_cnt(x, srcp, dstp, z128, z16)

  p2, r2, ic = pl.pallas_call(
      _tc1_body,
      grid=(_GRID,),
      in_specs=[
          _part_blk(128),
          _part_blk(16),
          _row_blk((128,)),
          _full_blk((128, 256)),
          _full_blk((1, 256)),
          _full_blk((128, 256)),
          _full_blk((256, 128)),
          _full_blk((256, 128)),
          _full_blk((1, 128)),
      ],
      out_specs=[_row_blk((128,)), _row_blk((128,)), _row_blk((8,))],
      out_shape=[
          jax.ShapeDtypeStruct((N_NODES, 128), jnp.float32),
          jax.ShapeDtypeStruct((N_NODES, 128), jnp.float32),
          jax.ShapeDtypeStruct((N_NODES, 8), jnp.float32),
      ],
  )(s1, cnt, x, W1_rel, b1.reshape(1, 256), W1_root, W2_rel, W2_root,
    b2.reshape(1, 128))

  (s2,) = _sc_agg_128(p2, srcp, dstp, z128)

  p3, r3 = pl.pallas_call(
      _tc2_body,
      grid=(_GRID,),
      in_specs=[
          _part_blk(128),
          _row_blk((128,)),
          _row_blk((8,)),
          _full_blk((128, 2)),
          _full_blk((128, 2)),
          _full_blk((128, 2)),
          _full_blk((128, 2)),
          _full_blk((1, 2)),
          _full_blk((1, 2)),
      ],
      out_specs=[_row_blk((16,)), _row_blk((16,))],
      out_shape=[
          jax.ShapeDtypeStruct((N_NODES, 16), jnp.float32),
          jax.ShapeDtypeStruct((N_NODES, 16), jnp.float32),
      ],
  )(s2, r2, ic, Wmu_rel, Wls_rel, Wmu_root, Wls_root,
    bmu.reshape(1, 2), bls.reshape(1, 2))

  (s3,) = _sc_agg_16(p3, srcp, dstp, z16)

  out = pl.pallas_call(
      _tc3_body,
      grid=(_GRID,),
      in_specs=[_part_blk(16), _row_blk((16,)), _row_blk((8,))],
      out_specs=_row_blk((16,)),
      out_shape=jax.ShapeDtypeStruct((N_NODES, 16), jnp.float32),
  )(s3, r3, ic)

  return out[:, 0:2], out[:, 2:4]
